# baseline (speedup 1.0000x reference)
import jax
import jax.numpy as jnp
from jax.experimental import pallas as pl
from jax.experimental.pallas import tpu as pltpu

B, N, F = 8, 576, 768
H = 8
M = 1024
D = F // H
EPS = 1e-10
BN = B * N
TN = 1152
T = BN // TN


def _prep_body(cb_ref, c2_ref, csum_ref):
    for h in range(H):
        c = cb_ref[h]
        cn = jnp.sqrt(jnp.sum(c * c, axis=1, keepdims=True))
        c2_ref[h] = c / jnp.maximum(cn, 1e-12)
        csum_ref[0, h, :] = jnp.sum(c, axis=0)


def _hvq_body(x_ref, c2_ref, csum_ref, out_ref, idx_ref, counts_ref, perp_ref):
    t = pl.program_id(0)

    @pl.when(t == 0)
    def _init():
        counts_ref[...] = jnp.zeros_like(counts_ref)

    x = x_ref[...]
    mi = jax.lax.broadcasted_iota(
        jnp.int32, (TN, M), 1).astype(jnp.float32)
    for h in range(H):
        q = x[:, h * D:(h + 1) * D]
        qn = jnp.sqrt(jnp.sum(q * q, axis=1, keepdims=True))
        q2 = q / jnp.maximum(qn, 1e-12)
        sim = jax.lax.dot_general(q2, c2_ref[h], (((1,), (1,)), ((), ())),
                                  preferred_element_type=jnp.float32)
        mx = jnp.max(sim, axis=1, keepdims=True)
        is_mx = sim >= mx
        idxh = jnp.min(jnp.where(is_mx, mi, float(M)), axis=1)
        idxi = idxh.astype(jnp.int32)
        for k in range(TN // N):
            idx_ref[k, h, :] = idxi[k * N:(k + 1) * N]
        counts_ref[h, :] = counts_ref[h, :] + jnp.sum(
            is_mx.astype(jnp.float32), axis=0)
        out_ref[:, h * D:(h + 1) * D] = jnp.broadcast_to(
            csum_ref[0, h, :][None, :], (TN, D))

    @pl.when(t == pl.num_programs(0) - 1)
    def _perp():
        mean = counts_ref[...] / float(BN)
        ent = -jnp.sum(mean * jnp.log(mean + EPS), axis=1, keepdims=True)
        perp_ref[...] = jnp.broadcast_to(jnp.exp(ent), perp_ref.shape)


def kernel(x, codebooks):
    x2 = x.reshape(BN, F)
    c2, csum = pl.pallas_call(
        _prep_body,
        in_specs=[pl.BlockSpec((H, M, D), lambda: (0, 0, 0))],
        out_specs=[
            pl.BlockSpec((H, M, D), lambda: (0, 0, 0)),
            pl.BlockSpec((1, H, D), lambda: (0, 0, 0)),
        ],
        out_shape=[
            jax.ShapeDtypeStruct((H, M, D), jnp.float32),
            jax.ShapeDtypeStruct((1, H, D), jnp.float32),
        ],
    )(codebooks)
    out2, idx, _counts, perp2 = pl.pallas_call(
        _hvq_body,
        grid=(T,),
        in_specs=[
            pl.BlockSpec((TN, F), lambda t: (t, 0)),
            pl.BlockSpec((H, M, D), lambda t: (0, 0, 0)),
            pl.BlockSpec((1, H, D), lambda t: (0, 0, 0)),
        ],
        out_specs=[
            pl.BlockSpec((TN, F), lambda t: (t, 0)),
            pl.BlockSpec((TN // N, H, N), lambda t: (t, 0, 0)),
            pl.BlockSpec((H, M), lambda t: (0, 0)),
            pl.BlockSpec((H, 128), lambda t: (0, 0)),
        ],
        out_shape=[
            jax.ShapeDtypeStruct((BN, F), jnp.float32),
            jax.ShapeDtypeStruct((B, H, N), jnp.int32),
            jax.ShapeDtypeStruct((H, M), jnp.float32),
            jax.ShapeDtypeStruct((H, 128), jnp.float32),
        ],
    )(x2, c2, csum)
    out = out2.reshape(B, N, F)
    codebook_indices = idx
    perp = perp2[:, 0]
    return (out, codebook_indices, perp)


# speedup vs baseline: 1.0483x; 1.0483x over previous
import jax
import jax.numpy as jnp
from jax.experimental import pallas as pl
from jax.experimental.pallas import tpu as pltpu

B, N, F = 8, 576, 768
H = 8
M = 1024
D = F // H
EPS = 1e-10
BN = B * N
TN = 1152
T = BN // TN


def _hvq_body(x_ref, cb_ref, out_ref, idx_ref, counts_ref, perp_ref,
              c2_ref, csum_ref):
    t = pl.program_id(0)

    @pl.when(t == 0)
    def _init():
        counts_ref[...] = jnp.zeros_like(counts_ref)
        for h in range(H):
            c = cb_ref[h]
            cn = jnp.sqrt(jnp.sum(c * c, axis=1, keepdims=True))
            c2_ref[h] = c / jnp.maximum(cn, 1e-12)
            csum_ref[0, h, :] = jnp.sum(c, axis=0)

    x = x_ref[...]
    mi = jax.lax.broadcasted_iota(
        jnp.int32, (TN, M), 1).astype(jnp.float32)
    for h in range(H):
        q = x[:, h * D:(h + 1) * D]
        qn = jnp.sqrt(jnp.sum(q * q, axis=1, keepdims=True))
        q2 = q / jnp.maximum(qn, 1e-12)
        sim = jax.lax.dot_general(q2, c2_ref[h], (((1,), (1,)), ((), ())),
                                  preferred_element_type=jnp.float32)
        mx = jnp.max(sim, axis=1, keepdims=True)
        is_mx = sim >= mx
        idxh = jnp.min(jnp.where(is_mx, mi, float(M)), axis=1)
        idxi = idxh.astype(jnp.int32)
        for k in range(TN // N):
            idx_ref[k, h, :] = idxi[k * N:(k + 1) * N]
        counts_ref[h, :] = counts_ref[h, :] + jnp.sum(
            is_mx.astype(jnp.float32), axis=0)
        out_ref[:, h * D:(h + 1) * D] = jnp.broadcast_to(
            csum_ref[0, h, :][None, :], (TN, D))

    @pl.when(t == pl.num_programs(0) - 1)
    def _perp():
        mean = counts_ref[...] / float(BN)
        ent = -jnp.sum(mean * jnp.log(mean + EPS), axis=1, keepdims=True)
        perp_ref[...] = jnp.broadcast_to(jnp.exp(ent), perp_ref.shape)


def kernel(x, codebooks):
    x2 = x.reshape(BN, F)
    out2, idx, _counts, perp2 = pl.pallas_call(
        _hvq_body,
        grid=(T,),
        in_specs=[
            pl.BlockSpec((TN, F), lambda t: (t, 0)),
            pl.BlockSpec((H, M, D), lambda t: (0, 0, 0)),
        ],
        out_specs=[
            pl.BlockSpec((TN, F), lambda t: (t, 0)),
            pl.BlockSpec((TN // N, H, N), lambda t: (t, 0, 0)),
            pl.BlockSpec((H, M), lambda t: (0, 0)),
            pl.BlockSpec((H, 128), lambda t: (0, 0)),
        ],
        out_shape=[
            jax.ShapeDtypeStruct((BN, F), jnp.float32),
            jax.ShapeDtypeStruct((B, H, N), jnp.int32),
            jax.ShapeDtypeStruct((H, M), jnp.float32),
            jax.ShapeDtypeStruct((H, 128), jnp.float32),
        ],
        scratch_shapes=[
            pltpu.VMEM((H, M, D), jnp.float32),
            pltpu.VMEM((1, H, D), jnp.float32),
        ],
    )(x2, codebooks)
    out = out2.reshape(B, N, F)
    codebook_indices = idx
    perp = perp2[:, 0]
    return (out, codebook_indices, perp)
